# baseline (speedup 1.0000x reference)
import functools

import jax
import jax.numpy as jnp
from jax import lax
from jax.experimental import pallas as pl
from jax.experimental.pallas import tpu as pltpu
from jax.experimental.pallas import tpu_sc as plsc

_TEMP = 2.0 / 3.0
_LIMIT_A = -0.1
_LIMIT_B = 1.1
_K = 256

_NC = 2
_NS = 16
_NW = _NC * _NS
_L = 16

_ROWS = 128
_COLS = 32768
_RPW = _ROWS // _NW
_NV = _COLS // _L


def _zs_kernel(x_ref, kz_ref, s_ref):
    xb = x_ref[...]
    u = (xb + 1.0) * (1.0 / _TEMP)
    z = jax.nn.sigmoid(u)
    r = z * (_LIMIT_B - _LIMIT_A) + _LIMIT_A
    s_ref[...] = jnp.clip(r, 0.0, 1.0)
    kz_ref[...] = jax.lax.bitcast_convert_type(z, jnp.int32)


def _sc_row_select(row_ref, hist_ref, hsum_ref, pexc_ref, kc, lane, ones):
    def hist_pass(nb, bucket_fn, mask_fn):
        def body(i, _):
            v = row_ref[pl.ds(i * _L, _L)]
            idx = lane * nb + bucket_fn(v)
            if mask_fn is None:
                plsc.addupdate_scatter(hist_ref, [idx], ones)
            else:
                plsc.addupdate_scatter(hist_ref, [idx], ones, mask=mask_fn(v))
            return 0
        lax.fori_loop(0, _NV, body, 0)

    def scan(nb, kcur):
        ngroups = nb // _L
        zeros = jnp.zeros((_L,), jnp.int32)

        def sum_body(j, acc):
            h = zeros
            for l in range(_L):
                sl = pl.ds(l * nb + j * _L, _L)
                h = h + hist_ref[sl]
                hist_ref[sl] = zeros
            hsum_ref[pl.ds(j * _L, _L)] = h
            return acc + h

        accv = lax.fori_loop(0, ngroups, sum_body, zeros)
        total = jnp.sum(accv)

        def pexc_body(j, carry):
            pe, cntv = carry
            h = hsum_ref[pl.ds(j * _L, _L)]
            cs = plsc.cumsum(h)
            pexc = pe + cs - h
            pexc_ref[pl.ds(j * _L, _L)] = pexc
            s_suffix = total - pexc
            cntv = cntv + jnp.where(s_suffix >= kcur, 1, 0).astype(jnp.int32)
            return pe + jnp.sum(h), cntv

        _, cntv = lax.fori_loop(0, ngroups, pexc_body,
                                (jnp.int32(0), zeros))
        b_sel = jnp.sum(cntv) - 1
        bidx = jnp.broadcast_to(b_sel, (_L,)).astype(jnp.int32)
        h_b = jnp.max(plsc.load_gather(hsum_ref, [bidx]))
        pexc_b = jnp.max(plsc.load_gather(pexc_ref, [bidx]))
        above = total - pexc_b - h_b
        return b_sel, kcur - above

    hist_pass(2048, lambda v: v >> 19, None)
    b1, k2 = scan(2048, kc)
    hist_pass(2048, lambda v: (v >> 8) & 0x7FF,
              lambda v: (v >> 19) == b1)
    b2, k3 = scan(2048, k2)
    p2 = (b1 << 11) | b2
    hist_pass(256, lambda v: v & 0xFF, lambda v: (v >> 8) == p2)
    b3, m = scan(256, k3)
    t = (p2 << 8) | b3
    return t, m


def _sc_tie_pass(row_ref, t, m, lane):
    def body(i, carry):
        cvec, acc = carry
        v = row_ref[pl.ds(i * _L, _L)]
        tie = v == t
        tiei = jnp.where(tie, 1, 0).astype(jnp.int32)
        pc = plsc.cumsum(tiei)
        tot = plsc.all_reduce_population_count(tie)
        sel = tie & ((cvec + pc) == m)
        acc = acc + jnp.where(sel, i * _L + lane, 0).astype(jnp.int32)
        return cvec + tot, acc

    zeros = jnp.zeros((_L,), jnp.int32)
    _, acc = lax.fori_loop(0, _NV, body, (zeros, zeros))
    return jnp.max(acc)


def _sc_select_body(kz_hbm, t_hbm, i_hbm, row_v, hist_v, hsum_v, pexc_v,
                    out_v):
    wid = lax.axis_index("s") * _NC + lax.axis_index("c")
    lane = lax.iota(jnp.int32, 16)
    ones = jnp.ones((_L,), jnp.int32)

    def zbody(i, _):
        hist_v[pl.ds(i * _L, _L)] = jnp.zeros((_L,), jnp.int32)
        return 0
    lax.fori_loop(0, _COLS // _L, zbody, 0)

    def row_body(j, carry):
        tvec, ivec = carry
        r = wid * _RPW + j
        pltpu.sync_copy(kz_hbm.at[r], row_v)
        t, m = _sc_row_select(row_v, hist_v, hsum_v, pexc_v,
                              jnp.int32(_K), lane, ones)
        istar = _sc_tie_pass(row_v, t, m, lane)
        tvec = jnp.where(lane == j, t, tvec)
        ivec = jnp.where(lane == j, istar, ivec)
        return tvec, ivec

    zeros = jnp.zeros((_L,), jnp.int32)
    tvec, ivec = lax.fori_loop(0, _RPW, row_body, (zeros, zeros))
    out_v[...] = tvec
    pltpu.sync_copy(out_v, t_hbm.at[wid])
    out_v[...] = ivec
    pltpu.sync_copy(out_v, i_hbm.at[wid])


_sc_select = functools.partial(
    pl.kernel,
    out_type=[
        jax.ShapeDtypeStruct((_NW, _L), jnp.int32),
        jax.ShapeDtypeStruct((_NW, _L), jnp.int32),
    ],
    mesh=plsc.VectorSubcoreMesh(core_axis_name="c", subcore_axis_name="s",
                                num_cores=_NC, num_subcores=_NS),
    compiler_params=pltpu.CompilerParams(needs_layout_passes=False),
    scratch_types=[
        pltpu.VMEM((_COLS,), jnp.int32),
        pltpu.VMEM((_COLS,), jnp.int32),
        pltpu.VMEM((2048,), jnp.int32),
        pltpu.VMEM((2048,), jnp.int32),
        pltpu.VMEM((_L,), jnp.int32),
    ],
)(_sc_select_body)


def _mask_kernel(kz_ref, t_ref, i_ref, mask_ref):
    kz = kz_ref[...]
    rows, cols = kz.shape
    t = t_ref[...]
    istar = i_ref[...]
    col = jax.lax.broadcasted_iota(jnp.int32, (rows, cols), 1)
    sel = (kz > t) | ((kz == t) & (col <= istar))
    mask_ref[...] = sel.astype(jnp.float32)


def kernel(x):
    n_rows, n_cols = x.shape
    block_rows = 32
    grid = (n_rows // block_rows,)
    blk = pl.BlockSpec((block_rows, n_cols), lambda i: (i, 0))
    sblk = pl.BlockSpec((block_rows, 1), lambda i: (i, 0))

    kz, s = pl.pallas_call(
        _zs_kernel,
        grid=grid,
        in_specs=[blk],
        out_specs=(blk, blk),
        out_shape=(
            jax.ShapeDtypeStruct((n_rows, n_cols), jnp.int32),
            jax.ShapeDtypeStruct((n_rows, n_cols), jnp.float32),
        ),
    )(x)

    t_w, i_w = _sc_select(kz)
    t = t_w[:, :_RPW].reshape(n_rows, 1)
    istar = i_w[:, :_RPW].reshape(n_rows, 1)

    mask = pl.pallas_call(
        _mask_kernel,
        grid=grid,
        in_specs=[blk, sblk, sblk],
        out_specs=blk,
        out_shape=jax.ShapeDtypeStruct((n_rows, n_cols), jnp.float32),
    )(kz, t, istar)
    return (mask, s)


# speedup vs baseline: 1.1501x; 1.1501x over previous
import functools

import jax
import jax.numpy as jnp
from jax import lax
from jax.experimental import pallas as pl
from jax.experimental.pallas import tpu as pltpu
from jax.experimental.pallas import tpu_sc as plsc

_TEMP = 2.0 / 3.0
_LIMIT_A = -0.1
_LIMIT_B = 1.1
_K = 256

_NC = 2
_NS = 16
_NW = _NC * _NS
_L = 16

_ROWS = 128
_COLS = 32768
_RPW = _ROWS // _NW
_NV = _COLS // _L


def _zs_kernel(x_ref, kz_ref, s_ref):
    xb = x_ref[...]
    u = (xb + 1.0) * (1.0 / _TEMP)
    z = jax.nn.sigmoid(u)
    r = z * (_LIMIT_B - _LIMIT_A) + _LIMIT_A
    s_ref[...] = jnp.clip(r, 0.0, 1.0)
    kz_ref[...] = jax.lax.bitcast_convert_type(z, jnp.int32)


def _sc_row_select(row_ref, hist_ref, hsum_ref, pexc_ref, kc, lane, ones):
    def hist_pass(nb, bucket_fn, mask_fn):
        unroll = 8

        def body(i, _):
            base = i * (_L * unroll)
            for u in range(unroll):
                v = row_ref[pl.ds(base + u * _L, _L)]
                idx = lane * nb + bucket_fn(v)
                if mask_fn is None:
                    plsc.addupdate_scatter(hist_ref, [idx], ones)
                else:
                    plsc.addupdate_scatter(hist_ref, [idx], ones,
                                           mask=mask_fn(v))
            return 0
        lax.fori_loop(0, _NV // unroll, body, 0)

    def scan(nb, kcur):
        ngroups = nb // _L
        zeros = jnp.zeros((_L,), jnp.int32)

        def sum_body(j, acc):
            h = zeros
            for l in range(_L):
                sl = pl.ds(l * nb + j * _L, _L)
                h = h + hist_ref[sl]
                hist_ref[sl] = zeros
            hsum_ref[pl.ds(j * _L, _L)] = h
            return acc + h

        accv = lax.fori_loop(0, ngroups, sum_body, zeros)
        total = jnp.sum(accv)

        def pexc_body(j, carry):
            pe, cntv = carry
            h = hsum_ref[pl.ds(j * _L, _L)]
            cs = plsc.cumsum(h)
            pexc = pe + cs - h
            pexc_ref[pl.ds(j * _L, _L)] = pexc
            s_suffix = total - pexc
            cntv = cntv + jnp.where(s_suffix >= kcur, 1, 0).astype(jnp.int32)
            return pe + jnp.sum(h), cntv

        _, cntv = lax.fori_loop(0, ngroups, pexc_body,
                                (jnp.int32(0), zeros))
        b_sel = jnp.sum(cntv) - 1
        bidx = jnp.broadcast_to(b_sel, (_L,)).astype(jnp.int32)
        h_b = jnp.max(plsc.load_gather(hsum_ref, [bidx]))
        pexc_b = jnp.max(plsc.load_gather(pexc_ref, [bidx]))
        above = total - pexc_b - h_b
        return b_sel, kcur - above

    hist_pass(2048, lambda v: v >> 19, None)
    b1, k2 = scan(2048, kc)
    hist_pass(2048, lambda v: (v >> 8) & 0x7FF,
              lambda v: (v >> 19) == b1)
    b2, k3 = scan(2048, k2)
    p2 = (b1 << 11) | b2
    hist_pass(256, lambda v: v & 0xFF, lambda v: (v >> 8) == p2)
    b3, m = scan(256, k3)
    t = (p2 << 8) | b3
    return t, m


def _sc_tie_pass(row_ref, t, m, lane):
    unroll = 8

    def body(i, carry):
        cvec, acc = carry
        base = i * (_L * unroll)
        for u in range(unroll):
            v = row_ref[pl.ds(base + u * _L, _L)]
            tie = v == t
            tiei = jnp.where(tie, 1, 0).astype(jnp.int32)
            pc = plsc.cumsum(tiei)
            tot = plsc.all_reduce_population_count(tie)
            sel = tie & ((cvec + pc) == m)
            acc = acc + jnp.where(sel, base + u * _L + lane,
                                  0).astype(jnp.int32)
            cvec = cvec + tot
        return cvec, acc

    zeros = jnp.zeros((_L,), jnp.int32)
    _, acc = lax.fori_loop(0, _NV // unroll, body, (zeros, zeros))
    return jnp.max(acc)


def _sc_select_body(kz_hbm, t_hbm, i_hbm, row_v, hist_v, hsum_v, pexc_v,
                    out_v):
    wid = lax.axis_index("s") * _NC + lax.axis_index("c")
    lane = lax.iota(jnp.int32, 16)
    ones = jnp.ones((_L,), jnp.int32)

    def zbody(i, _):
        for u in range(8):
            hist_v[pl.ds((i * 8 + u) * _L, _L)] = jnp.zeros((_L,), jnp.int32)
        return 0
    lax.fori_loop(0, _COLS // (_L * 8), zbody, 0)

    def row_body(j, carry):
        tvec, ivec = carry
        r = wid * _RPW + j
        pltpu.sync_copy(kz_hbm.at[r], row_v)
        t, m = _sc_row_select(row_v, hist_v, hsum_v, pexc_v,
                              jnp.int32(_K), lane, ones)
        istar = _sc_tie_pass(row_v, t, m, lane)
        tvec = jnp.where(lane == j, t, tvec)
        ivec = jnp.where(lane == j, istar, ivec)
        return tvec, ivec

    zeros = jnp.zeros((_L,), jnp.int32)
    tvec, ivec = lax.fori_loop(0, _RPW, row_body, (zeros, zeros))
    out_v[...] = tvec
    pltpu.sync_copy(out_v, t_hbm.at[wid])
    out_v[...] = ivec
    pltpu.sync_copy(out_v, i_hbm.at[wid])


_sc_select = functools.partial(
    pl.kernel,
    out_type=[
        jax.ShapeDtypeStruct((_NW, _L), jnp.int32),
        jax.ShapeDtypeStruct((_NW, _L), jnp.int32),
    ],
    mesh=plsc.VectorSubcoreMesh(core_axis_name="c", subcore_axis_name="s",
                                num_cores=_NC, num_subcores=_NS),
    compiler_params=pltpu.CompilerParams(needs_layout_passes=False),
    scratch_types=[
        pltpu.VMEM((_COLS,), jnp.int32),
        pltpu.VMEM((_COLS,), jnp.int32),
        pltpu.VMEM((2048,), jnp.int32),
        pltpu.VMEM((2048,), jnp.int32),
        pltpu.VMEM((_L,), jnp.int32),
    ],
)(_sc_select_body)


def _mask_kernel(kz_ref, t_ref, i_ref, mask_ref):
    kz = kz_ref[...]
    rows, cols = kz.shape
    t = t_ref[...]
    istar = i_ref[...]
    col = jax.lax.broadcasted_iota(jnp.int32, (rows, cols), 1)
    sel = (kz > t) | ((kz == t) & (col <= istar))
    mask_ref[...] = sel.astype(jnp.float32)


def kernel(x):
    n_rows, n_cols = x.shape
    block_rows = 32
    grid = (n_rows // block_rows,)
    blk = pl.BlockSpec((block_rows, n_cols), lambda i: (i, 0))
    sblk = pl.BlockSpec((block_rows, 1), lambda i: (i, 0))

    kz, s = pl.pallas_call(
        _zs_kernel,
        grid=grid,
        in_specs=[blk],
        out_specs=(blk, blk),
        out_shape=(
            jax.ShapeDtypeStruct((n_rows, n_cols), jnp.int32),
            jax.ShapeDtypeStruct((n_rows, n_cols), jnp.float32),
        ),
    )(x)

    t_w, i_w = _sc_select(kz)
    t = t_w[:, :_RPW].reshape(n_rows, 1)
    istar = i_w[:, :_RPW].reshape(n_rows, 1)

    mask = pl.pallas_call(
        _mask_kernel,
        grid=grid,
        in_specs=[blk, sblk, sblk],
        out_specs=blk,
        out_shape=jax.ShapeDtypeStruct((n_rows, n_cols), jnp.float32),
    )(kz, t, istar)
    return (mask, s)


# speedup vs baseline: 1.4906x; 1.2961x over previous
import functools

import jax
import jax.numpy as jnp
from jax import lax
from jax.experimental import pallas as pl
from jax.experimental.pallas import tpu as pltpu
from jax.experimental.pallas import tpu_sc as plsc

_TEMP = 2.0 / 3.0
_LIMIT_A = -0.1
_LIMIT_B = 1.1
_K = 256

_NC = 2
_NS = 16
_NW = _NC * _NS
_L = 16

_ROWS = 128
_COLS = 32768
_RPW = _ROWS // _NW
_NV = _COLS // _L

_TC_BITS = 12
_SC_BITS = 30 - _TC_BITS
_BUCKET = 1 << _SC_BITS


def _rowsum(v):
    n = v.shape[1]
    c = n // 4
    p = [
        jnp.sum(v[:, i * c:(i + 1) * c].astype(jnp.int32), axis=1,
                keepdims=True)
        for i in range(4)
    ]
    return (p[0] + p[1]) + (p[2] + p[3])


def _zs_kernel(x_ref, kz_ref, s_ref, tb_ref, k2_ref):
    xb = x_ref[...]
    rows, _ = xb.shape
    u = (xb + 1.0) * (1.0 / _TEMP)
    z = jax.nn.sigmoid(u)
    r = z * (_LIMIT_B - _LIMIT_A) + _LIMIT_A
    s_ref[...] = jnp.clip(r, 0.0, 1.0)
    key = jax.lax.bitcast_convert_type(z, jnp.int32)
    kz_ref[...] = key

    p = jnp.zeros((rows, 1), dtype=jnp.int32)

    def body(i, carry):
        p, step = carry
        cand = p + step
        cnt = _rowsum(key >= cand)
        return jnp.where(cnt >= _K, cand, p), step >> 1

    p, _ = jax.lax.fori_loop(0, _TC_BITS, body, (p, jnp.int32(1 << 29)))
    above = _rowsum(key >= p + _BUCKET)
    tb_ref[...] = p
    k2_ref[...] = _K - above


def _sc_select_body(kz_hbm, tbk_hbm, t_hbm, i_hbm, row_v, ck_v, cc_v, io_v):
    wid = lax.axis_index("s") * _NC + lax.axis_index("c")
    lane = lax.iota(jnp.int32, _L)

    pltpu.sync_copy(tbk_hbm.at[wid], io_v)
    params = io_v[...]

    def row_body(j, carry):
        tvec, ivec = carry
        r = wid * _RPW + j
        pltpu.sync_copy(kz_hbm.at[r], row_v)
        tb = jnp.max(jnp.where(lane == j, params, 0))
        k2 = jnp.max(jnp.where(lane == j + _RPW, params, 0))

        unroll = 8

        tbs = tb >> _SC_BITS

        def cbody(i, off):
            base = i * (_L * unroll)
            for u in range(unroll):
                v = row_v[pl.ds(base + u * _L, _L)]
                m = (v >> _SC_BITS) == tbs
                pc = plsc.cumsum(jnp.where(m, 1, 0).astype(jnp.int32))
                idx = off + pc - 1
                plsc.store_scatter(ck_v, [idx], v, mask=m)
                plsc.store_scatter(cc_v, [idx], base + u * _L + lane, mask=m)
                off = off + jnp.max(plsc.all_reduce_population_count(m))
            return off

        c1 = lax.fori_loop(0, _NV // unroll, cbody, jnp.int32(0))
        plsc.store_scatter(ck_v, [c1 + lane], jnp.full((_L,), -1, jnp.int32))
        nch = (c1 + _L - 1) >> 4

        def count_ge(cand):
            def body(i, acc):
                v = ck_v[pl.ds(i * _L, _L)]
                return acc + jnp.max(
                    plsc.all_reduce_population_count(v >= cand))
            return lax.fori_loop(0, nch, body, jnp.int32(0))

        def bis(b, carry):
            p, step = carry
            cand = p + step
            return jnp.where(count_ge(cand) >= k2, cand, p), step >> 1

        (t, _) = lax.fori_loop(0, _SC_BITS, bis,
                               (tb, jnp.int32(1 << (_SC_BITS - 1))))
        m = k2 - count_ge(t + 1)

        def tie_body(i, carry):
            cvec, acc = carry
            v = ck_v[pl.ds(i * _L, _L)]
            cols = cc_v[pl.ds(i * _L, _L)]
            tie = v == t
            pc = plsc.cumsum(jnp.where(tie, 1, 0).astype(jnp.int32))
            tot = plsc.all_reduce_population_count(tie)
            sel = tie & ((cvec + pc) == m)
            acc = acc + jnp.where(sel, cols, 0).astype(jnp.int32)
            return cvec + tot, acc

        zeros = jnp.zeros((_L,), jnp.int32)
        _, acc = lax.fori_loop(0, nch, tie_body, (zeros, zeros))
        istar = jnp.max(acc)

        tvec = jnp.where(lane == j, t, tvec)
        ivec = jnp.where(lane == j, istar, ivec)
        return tvec, ivec

    zeros = jnp.zeros((_L,), jnp.int32)
    tvec, ivec = lax.fori_loop(0, _RPW, row_body, (zeros, zeros))
    io_v[...] = tvec
    pltpu.sync_copy(io_v, t_hbm.at[wid])
    io_v[...] = ivec
    pltpu.sync_copy(io_v, i_hbm.at[wid])


_sc_select = functools.partial(
    pl.kernel,
    out_type=[
        jax.ShapeDtypeStruct((_NW, _L), jnp.int32),
        jax.ShapeDtypeStruct((_NW, _L), jnp.int32),
    ],
    mesh=plsc.VectorSubcoreMesh(core_axis_name="c", subcore_axis_name="s",
                                num_cores=_NC, num_subcores=_NS),
    compiler_params=pltpu.CompilerParams(needs_layout_passes=False),
    scratch_types=[
        pltpu.VMEM((_COLS,), jnp.int32),
        pltpu.VMEM((_COLS + _L,), jnp.int32),
        pltpu.VMEM((_COLS + _L,), jnp.int32),
        pltpu.VMEM((_L,), jnp.int32),
    ],
)(_sc_select_body)


def _mask_kernel(kz_ref, t_ref, i_ref, mask_ref):
    kz = kz_ref[...]
    rows, cols = kz.shape
    t = t_ref[...]
    istar = i_ref[...]
    col = jax.lax.broadcasted_iota(jnp.int32, (rows, cols), 1)
    sel = (kz > t) | ((kz == t) & (col <= istar))
    mask_ref[...] = sel.astype(jnp.float32)


def kernel(x):
    n_rows, n_cols = x.shape
    block_rows = 32
    grid = (n_rows // block_rows,)
    blk = pl.BlockSpec((block_rows, n_cols), lambda i: (i, 0))
    sblk = pl.BlockSpec((block_rows, 1), lambda i: (i, 0))

    kz, s, tb, k2 = pl.pallas_call(
        _zs_kernel,
        grid=grid,
        in_specs=[blk],
        out_specs=(blk, blk, sblk, sblk),
        out_shape=(
            jax.ShapeDtypeStruct((n_rows, n_cols), jnp.int32),
            jax.ShapeDtypeStruct((n_rows, n_cols), jnp.float32),
            jax.ShapeDtypeStruct((n_rows, 1), jnp.int32),
            jax.ShapeDtypeStruct((n_rows, 1), jnp.int32),
        ),
    )(x)

    tbk = jnp.concatenate(
        [tb.reshape(_NW, _RPW), k2.reshape(_NW, _RPW),
         jnp.zeros((_NW, _L - 2 * _RPW), jnp.int32)], axis=1)

    t_w, i_w = _sc_select(kz, tbk)
    t = t_w[:, :_RPW].reshape(n_rows, 1)
    istar = i_w[:, :_RPW].reshape(n_rows, 1)

    mask = pl.pallas_call(
        _mask_kernel,
        grid=grid,
        in_specs=[blk, sblk, sblk],
        out_specs=blk,
        out_shape=jax.ShapeDtypeStruct((n_rows, n_cols), jnp.float32),
    )(kz, t, istar)
    return (mask, s)
